# Initial kernel scaffold; baseline (speedup 1.0000x reference)
import jax
import jax.numpy as jnp
from jax.experimental import pallas as pl


def kernel(x, rows, cols, vals, W, b):
    raise NotImplementedError("write your pallas kernel here")


# speedup vs baseline: 5.2107x; 5.2107x over previous
import functools

import jax
import jax.numpy as jnp
from jax import lax
from jax.experimental import pallas as pl
from jax.experimental.pallas import tpu as pltpu
from jax.experimental.pallas import tpu_sc as plsc

N = 10000
NPAD = 10240
D = 128
NC = 2
NS = 16
NW = NC * NS
CHUNK = 128
ROWS_PER_TILE = NPAD // NS


def _sc_spmm(nchunk):
    mesh = plsc.VectorSubcoreMesh(core_axis_name="c", subcore_axis_name="s")

    @functools.partial(
        pl.kernel,
        out_type=jax.ShapeDtypeStruct((NC, NPAD, D), jnp.float32),
        mesh=mesh,
        scratch_types=[
            pltpu.VMEM((nchunk, CHUNK), jnp.int32),
            pltpu.VMEM((nchunk, CHUNK), jnp.int32),
            pltpu.VMEM((nchunk, CHUNK), jnp.float32),
            pltpu.VMEM((CHUNK, D), jnp.float32),
            pltpu.VMEM_SHARED((NPAD, D), jnp.float32),
            pltpu.SemaphoreType.DMA,
        ],
    )
    def k(x_hbm, cols_hbm, rows_hbm, vals_hbm, zeros_hbm, z_hbm,
          cols_v, rows_v, vals_v, g_v, acc, sem):
        cid = lax.axis_index("c")
        sid = lax.axis_index("s")
        wid = cid * NS + sid

        pltpu.sync_copy(cols_hbm.at[wid], cols_v)
        pltpu.sync_copy(rows_hbm.at[wid], rows_v)
        pltpu.sync_copy(vals_hbm.at[wid], vals_v)
        pltpu.sync_copy(zeros_hbm,
                        acc.at[pl.ds(sid * ROWS_PER_TILE, ROWS_PER_TILE)])
        plsc.subcore_barrier()

        @pl.loop(0, nchunk)
        def _chunk(j):
            pltpu.async_copy(x_hbm.at[cols_v.at[j]], g_v, sem).wait()

            @pl.loop(0, CHUNK // 16)
            def _scale(bgrp):
                v_vec = vals_v[j, pl.ds(bgrp * 16, 16)]
                for i in range(16):
                    v = v_vec[i]
                    e = bgrp * 16 + i
                    for kk in range(D // 16):
                        sl = pl.ds(kk * 16, 16)
                        g_v[e, sl] = g_v[e, sl] * v

            pltpu.sync_copy(g_v, acc.at[rows_v.at[j]], add=True)

        plsc.subcore_barrier()
        pltpu.sync_copy(acc.at[pl.ds(sid * ROWS_PER_TILE, ROWS_PER_TILE)],
                        z_hbm.at[cid, pl.ds(sid * ROWS_PER_TILE, ROWS_PER_TILE)])

    return k


def _tc_linear_body(z_ref, wt_ref, b_ref, o_ref):
    zsum = z_ref[0] + z_ref[1]
    o_ref[...] = (
        jnp.dot(zsum, wt_ref[...], preferred_element_type=jnp.float32)
        + b_ref[...]
    )


def _tc_linear(z, wt, b2d):
    rows_blk = 1000
    return pl.pallas_call(
        _tc_linear_body,
        grid=(N // rows_blk,),
        in_specs=[
            pl.BlockSpec((NC, rows_blk, D), lambda i: (0, i, 0)),
            pl.BlockSpec((D, D), lambda i: (0, 0)),
            pl.BlockSpec((1, D), lambda i: (0, 0)),
        ],
        out_specs=pl.BlockSpec((rows_blk, D), lambda i: (i, 0)),
        out_shape=jax.ShapeDtypeStruct((N, D), jnp.float32),
    )(z, wt, b2d)


def kernel(x, rows, cols, vals, W, b):
    nnz = rows.shape[0]
    per_worker = -(-nnz // (NW * CHUNK)) * CHUNK
    nchunk = per_worker // CHUNK
    pad = NW * per_worker - nnz

    rows_i = jnp.pad(rows.astype(jnp.int32), (0, pad)).reshape(NW, nchunk, CHUNK)
    cols_i = jnp.pad(cols.astype(jnp.int32), (0, pad)).reshape(NW, nchunk, CHUNK)
    vals_f = jnp.pad(vals, (0, pad)).reshape(NW, nchunk, CHUNK)
    zeros = jnp.zeros((NPAD // NS, D), jnp.float32)

    z = _sc_spmm(nchunk)(x, cols_i, rows_i, vals_f, zeros)
    return _tc_linear(z, W.T, b.reshape(1, D))
